# Initial kernel scaffold; baseline (speedup 1.0000x reference)
import jax
import jax.numpy as jnp
from jax.experimental import pallas as pl


def kernel(obs, p_hidden, s_hidden, batch_size, Wq, Wk, Wv, fc_W, fc_b):
    raise NotImplementedError("write your pallas kernel here")


# speedup vs baseline: 1.2555x; 1.2555x over previous
import jax
import jax.numpy as jnp
from jax.experimental import pallas as pl

N_P = 8
N_S = 64
COMM_RANGE = 0.3
HID = 256


def _body(px_ref, py_ref, sx_ref, sy_ref, ph_ref, s_ref,
          wq_ref, wk_ref, wv_ref, fcw_ref, fcb_ref,
          h_out_ref, alpha_ref):
    ph = ph_ref[...]
    s = s_ref[...]
    R = ph.shape[0]
    C = s.shape[0]

    q = jnp.dot(ph, wq_ref[...], preferred_element_type=jnp.float32)
    k = jnp.dot(s, wk_ref[...], preferred_element_type=jnp.float32)
    v = jnp.dot(s, wv_ref[...], preferred_element_type=jnp.float32)

    e = jax.lax.dot_general(q, k, (((1,), (1,)), ((), ())),
                            preferred_element_type=jnp.float32)
    e = e * (1.0 / jnp.sqrt(jnp.float32(HID)))

    dx = jnp.abs(px_ref[...] - sx_ref[...])
    dy = jnp.abs(py_ref[...] - sy_ref[...])
    near = jnp.maximum(dx, dy) <= COMM_RANGE
    rowb = jax.lax.broadcasted_iota(jnp.int32, (R, C), 0) // N_P
    colb = jax.lax.broadcasted_iota(jnp.int32, (R, C), 1) // N_S
    mask = near & (rowb == colb)

    e = jnp.where(mask, e, -1e30)
    m = jnp.max(e, axis=1, keepdims=True)
    ex = jnp.exp(e - m)
    a = ex / jnp.sum(ex, axis=1, keepdims=True)
    a = jnp.where(mask, a, 0.0)
    has_vis = jnp.any(mask, axis=1, keepdims=True)

    attn = jnp.dot(a, v, preferred_element_type=jnp.float32)
    h = jnp.where(has_vis, attn, ph)
    h_out_ref[...] = jnp.dot(h, fcw_ref[...],
                             preferred_element_type=jnp.float32) + fcb_ref[...]

    sel = (jax.lax.broadcasted_iota(jnp.int32, (C, N_S), 0) % N_S ==
           jax.lax.broadcasted_iota(jnp.int32, (C, N_S), 1))
    alpha_ref[...] = jnp.dot(a, sel.astype(jnp.float32),
                             preferred_element_type=jnp.float32)


def kernel(obs, p_hidden, s_hidden, batch_size, Wq, Wk, Wv, fc_W, fc_b):
    B = p_hidden.shape[0] // N_P
    px = obs[:, :N_P, 0].reshape(B * N_P, 1)
    py = obs[:, :N_P, 1].reshape(B * N_P, 1)
    sx = obs[:, N_P:, 0].reshape(1, B * N_S)
    sy = obs[:, N_P:, 1].reshape(1, B * N_S)
    s_flat = s_hidden.reshape(B * N_S, HID)
    fc_b2 = fc_b.reshape(1, HID)

    h_out, alpha = pl.pallas_call(
        _body,
        out_shape=[
            jax.ShapeDtypeStruct((B * N_P, HID), jnp.float32),
            jax.ShapeDtypeStruct((B * N_P, N_S), jnp.float32),
        ],
    )(px, py, sx, sy, p_hidden, s_flat, Wq, Wk, Wv, fc_W, fc_b2)
    return h_out.reshape(B, N_P, HID), alpha.reshape(B, N_P, N_S)


# speedup vs baseline: 1.4973x; 1.1926x over previous
import jax
import jax.numpy as jnp
from jax.experimental import pallas as pl

N_P = 8
N_S = 64
COMM_RANGE = 0.3
HID = 256


def _body(obs_ref, ph_ref, s_ref, wq_ref, wk_ref, wv_ref, fcw_ref, fcb_ref,
          h_out_ref, alpha_ref):
    ph = ph_ref[...]
    s = s_ref[...]
    R = ph.shape[0]
    C = s.shape[0]
    B = R // N_P

    ob = obs_ref[...]
    xc = ob[:, 0:1].reshape(B, N_P + N_S, 1)
    yc = ob[:, 1:2].reshape(B, N_P + N_S, 1)
    px = xc[:, :N_P, :].reshape(R, 1)
    py = yc[:, :N_P, :].reshape(R, 1)
    sx = xc[:, N_P:, :].reshape(C, 1)
    sy = yc[:, N_P:, :].reshape(C, 1)
    pxt = px.reshape(1, R)
    pyt = py.reshape(1, R)

    q = jnp.dot(ph, wq_ref[...], preferred_element_type=jnp.float32)
    k = jnp.dot(s, wk_ref[...], preferred_element_type=jnp.float32)
    v = jnp.dot(s, wv_ref[...], preferred_element_type=jnp.float32)

    et = jax.lax.dot_general(k, q, (((1,), (1,)), ((), ())),
                             preferred_element_type=jnp.float32)
    et = et * (1.0 / jnp.sqrt(jnp.float32(HID)))

    dx = jnp.abs(sx - pxt)
    dy = jnp.abs(sy - pyt)
    near = jnp.maximum(dx, dy) <= COMM_RANGE
    rowb = jax.lax.broadcasted_iota(jnp.int32, (C, R), 0) // N_S
    colb = jax.lax.broadcasted_iota(jnp.int32, (C, R), 1) // N_P
    maskt = near & (rowb == colb)

    et = jnp.where(maskt, et, -1e30)
    m = jnp.max(et, axis=0, keepdims=True)
    ex = jnp.exp(et - m)
    at = ex / jnp.sum(ex, axis=0, keepdims=True)
    at = jnp.where(maskt, at, 0.0)
    ones_c = jnp.ones((C, 1), jnp.float32)
    vis_count = jax.lax.dot_general(maskt.astype(jnp.float32), ones_c,
                                    (((0,), (0,)), ((), ())),
                                    preferred_element_type=jnp.float32)
    has_vis_col = vis_count > 0.0

    attn = jax.lax.dot_general(at, v, (((0,), (0,)), ((), ())),
                               preferred_element_type=jnp.float32)
    h = jnp.where(has_vis_col, attn, ph)
    h_out_ref[...] = jnp.dot(h, fcw_ref[...],
                             preferred_element_type=jnp.float32) + fcb_ref[...]

    sel = (jax.lax.broadcasted_iota(jnp.int32, (N_S, C), 1) % N_S ==
           jax.lax.broadcasted_iota(jnp.int32, (N_S, C), 0))
    alpha_t = jnp.dot(sel.astype(jnp.float32), at,
                      preferred_element_type=jnp.float32)
    alpha_ref[...] = alpha_t.T


def kernel(obs, p_hidden, s_hidden, batch_size, Wq, Wk, Wv, fc_W, fc_b):
    B = p_hidden.shape[0] // N_P
    obs2d = obs.reshape(B * (N_P + N_S), 8)
    s_flat = s_hidden.reshape(B * N_S, HID)
    fc_b2 = fc_b.reshape(1, HID)

    h_out, alpha = pl.pallas_call(
        _body,
        out_shape=[
            jax.ShapeDtypeStruct((B * N_P, HID), jnp.float32),
            jax.ShapeDtypeStruct((B * N_P, N_S), jnp.float32),
        ],
    )(obs2d, p_hidden, s_flat, Wq, Wk, Wv, fc_W, fc_b2)
    return h_out.reshape(B, N_P, HID), alpha.reshape(B, N_P, N_S)
